# baseline (speedup 1.0000x reference)
import jax
import jax.numpy as jnp
from jax import lax
from jax.experimental import pallas as pl
from jax.experimental.pallas import tpu as pltpu
from jax.experimental.pallas import tpu_sc as plsc

_B = 16384
_D = 64
_V = 100000
_NC = 2
_NS = 16
_NW = _NC * _NS
_ROUNDS = _D // _NW
_LANES = 16
_FCH = 4096
_NFCH = _B // _FCH


def _cl_body(feat_hbm, lab_hbm, cent_hbm, out_hbm, row_v, feat_a, feat_b,
             lab_v, acc_v, sem_r, sem_fa, sem_fb, sem_l):
    c = lax.axis_index("c")
    s = lax.axis_index("s")
    wid = s * _NC + c

    feats = (feat_a, feat_b)
    feat_sems = (sem_fa, sem_fb)

    def comp(r):
        return r * _NW + wid

    def feat_start(q):
        r, fc = divmod(q, _NFCH)
        return pltpu.async_copy(
            feat_hbm.at[comp(r), pl.ds(fc * _FCH, _FCH)],
            feats[q % 2], feat_sems[q % 2])

    def row_start(r):
        return pltpu.async_copy(cent_hbm.at[comp(r)], row_v, sem_r)

    lab_cp = pltpu.async_copy(lab_hbm, lab_v, sem_l)
    row_cp = row_start(0)
    feat_cp = feat_start(0)
    lab_cp.wait()

    zero = jnp.zeros((_LANES,), jnp.float32)
    accs = (zero, zero, zero, zero)
    for r in range(_ROUNDS):
        row_cp.wait()
        for fc in range(_NFCH):
            q = r * _NFCH + fc
            next_feat_cp = feat_start(q + 1) \
                if q + 1 < _ROUNDS * _NFCH else None
            feat_cp.wait()
            feat_v = feats[q % 2]

            def group_step(j, accs, fc=fc, feat_v=feat_v):
                out = []
                for k in range(4):
                    off = fc * _FCH + (j * 4 + k) * _LANES
                    idx = lab_v[pl.ds(off, _LANES)]
                    g = plsc.load_gather(row_v, [idx])
                    f = feat_v[pl.ds((j * 4 + k) * _LANES, _LANES)]
                    d = f - g
                    out.append(accs[k] + d * d)
                return tuple(out)

            accs = plsc.parallel_loop(
                0, _FCH // (4 * _LANES), carry=accs, unroll=4)(group_step)
            feat_cp = next_feat_cp
        if r + 1 < _ROUNDS:
            row_cp = row_start(r + 1)

    acc = (accs[0] + accs[1]) + (accs[2] + accs[3])
    acc_v[...] = acc * (1.0 / _B)
    pltpu.sync_copy(acc_v, out_hbm.at[wid])


@jax.jit
def kernel(features, labels, centers):
    labels_i = labels.astype(jnp.int32)
    feat_t = features.T
    cent_t = centers.T
    mesh = plsc.VectorSubcoreMesh(core_axis_name="c", subcore_axis_name="s")
    partials = pl.kernel(
        _cl_body,
        out_type=jax.ShapeDtypeStruct((_NW, _LANES), jnp.float32),
        mesh=mesh,
        scratch_types=[
            pltpu.VMEM((_V,), jnp.float32),
            pltpu.VMEM((_FCH,), jnp.float32),
            pltpu.VMEM((_FCH,), jnp.float32),
            pltpu.VMEM((_B,), jnp.int32),
            pltpu.VMEM((_LANES,), jnp.float32),
            pltpu.SemaphoreType.DMA,
            pltpu.SemaphoreType.DMA,
            pltpu.SemaphoreType.DMA,
            pltpu.SemaphoreType.DMA,
        ],
        compiler_params=pltpu.CompilerParams(use_tc_tiling_on_sc=True,
                                             needs_layout_passes=False),
    )(feat_t, labels_i, cent_t)
    return jnp.sum(partials)


# speedup vs baseline: 1.0323x; 1.0323x over previous
import jax
import jax.numpy as jnp
from jax import lax
from jax.experimental import pallas as pl
from jax.experimental.pallas import tpu as pltpu
from jax.experimental.pallas import tpu_sc as plsc

_B = 16384
_D = 64
_V = 100000
_NC = 2
_NS = 16
_NW = _NC * _NS
_ROUNDS = _D // _NW
_LANES = 16
_FCH = 4096
_NFCH = _B // _FCH


def _cl_body(feat_hbm, lab_hbm, cent_hbm, out_hbm, row_v, feat_a, feat_b,
             lab_v, acc_v, sem_r, sem_fa, sem_fb, sem_l):
    c = lax.axis_index("c")
    s = lax.axis_index("s")
    wid = s * _NC + c

    feats = (feat_a, feat_b)
    feat_sems = (sem_fa, sem_fb)

    def comp(r):
        return r * _NW + wid

    def feat_start(q):
        r, fc = divmod(q, _NFCH)
        return pltpu.async_copy(
            feat_hbm.at[comp(r), pl.ds(fc * _FCH, _FCH)],
            feats[q % 2], feat_sems[q % 2])

    def row_start(r):
        return pltpu.async_copy(cent_hbm.at[comp(r)], row_v, sem_r)

    lab_cp = pltpu.async_copy(lab_hbm, lab_v, sem_l)
    row_cp = row_start(0)
    feat_cp = feat_start(0)
    lab_cp.wait()

    zero = jnp.zeros((_LANES,), jnp.float32)
    accs = (zero, zero, zero, zero)
    for r in range(_ROUNDS):
        row_cp.wait()
        for fc in range(_NFCH):
            q = r * _NFCH + fc
            next_feat_cp = feat_start(q + 1) \
                if q + 1 < _ROUNDS * _NFCH else None
            feat_cp.wait()
            feat_v = feats[q % 2]

            def group_step(j, accs, fc=fc, feat_v=feat_v):
                out = []
                for k in range(4):
                    off = fc * _FCH + (j * 4 + k) * _LANES
                    idx = lab_v[pl.ds(off, _LANES)]
                    g = plsc.load_gather(row_v, [idx])
                    f = feat_v[pl.ds((j * 4 + k) * _LANES, _LANES)]
                    d = f - g
                    out.append(accs[k] + d * d)
                return tuple(out)

            accs = plsc.parallel_loop(
                0, _FCH // (4 * _LANES), carry=accs, unroll=1)(group_step)
            feat_cp = next_feat_cp
        if r + 1 < _ROUNDS:
            row_cp = row_start(r + 1)

    acc = (accs[0] + accs[1]) + (accs[2] + accs[3])
    acc_v[...] = acc * (1.0 / _B)
    pltpu.sync_copy(acc_v, out_hbm.at[wid])


@jax.jit
def kernel(features, labels, centers):
    labels_i = labels.astype(jnp.int32)
    feat_t = features.T
    cent_t = centers.T
    mesh = plsc.VectorSubcoreMesh(core_axis_name="c", subcore_axis_name="s")
    partials = pl.kernel(
        _cl_body,
        out_type=jax.ShapeDtypeStruct((_NW, _LANES), jnp.float32),
        mesh=mesh,
        scratch_types=[
            pltpu.VMEM((_V,), jnp.float32),
            pltpu.VMEM((_FCH,), jnp.float32),
            pltpu.VMEM((_FCH,), jnp.float32),
            pltpu.VMEM((_B,), jnp.int32),
            pltpu.VMEM((_LANES,), jnp.float32),
            pltpu.SemaphoreType.DMA,
            pltpu.SemaphoreType.DMA,
            pltpu.SemaphoreType.DMA,
            pltpu.SemaphoreType.DMA,
        ],
        compiler_params=pltpu.CompilerParams(use_tc_tiling_on_sc=True,
                                             needs_layout_passes=False),
    )(feat_t, labels_i, cent_t)
    return jnp.sum(partials)
